# Initial kernel scaffold; baseline (speedup 1.0000x reference)
import jax
import jax.numpy as jnp
from jax.experimental import pallas as pl


def kernel(A, input, init, final):
    raise NotImplementedError("write your pallas kernel here")


# speedup vs baseline: 1.3090x; 1.3090x over previous
import functools
import jax
import jax.numpy as jnp
from jax import lax
from jax.experimental import pallas as pl
from jax.experimental.pallas import tpu as pltpu
from jax.experimental.pallas import tpu_sc as plsc

NC, NS = 2, 16
NW = NC * NS


def _chunk_sizes(n):
    chunks = []
    left = n
    while left > 0:
        c = min(128, left)
        if left > c:
            c -= c % 8
        chunks.append(c)
        left -= c
    return chunks


def _make_gather(QV, Q, B):
    b_per_w = B // NW
    chunks = _chunk_sizes(b_per_w)
    mesh = plsc.VectorSubcoreMesh(core_axis_name="c", subcore_axis_name="s")

    scratch = []
    for c in chunks:
        scratch.append(pltpu.VMEM((c,), jnp.int32))
        scratch.append(pltpu.VMEM((c, Q), jnp.float32))
        scratch.append(pltpu.SemaphoreType.DMA)

    @functools.partial(
        pl.kernel,
        out_type=jax.ShapeDtypeStruct((B, Q), jnp.float32),
        mesh=mesh,
        scratch_types=scratch,
        compiler_params=pltpu.CompilerParams(use_tc_tiling_on_sc=False),
    )
    def gather(table_hbm, idx_hbm, out_hbm, *bufs):
        wid = lax.axis_index("s") * NC + lax.axis_index("c")
        base = wid * b_per_w
        copies = []
        off = 0
        for k, c in enumerate(chunks):
            idx_v, rows_v, sem = bufs[3 * k], bufs[3 * k + 1], bufs[3 * k + 2]
            pltpu.sync_copy(idx_hbm.at[pl.ds(base + off, c)], idx_v)
            copies.append(pltpu.async_copy(table_hbm.at[idx_v], rows_v, sem))
            off += c
        off = 0
        for k, c in enumerate(chunks):
            rows_v = bufs[3 * k + 1]
            copies[k].wait()
            pltpu.sync_copy(rows_v, out_hbm.at[pl.ds(base + off, c)])
            off += c

    return gather


def _make_chain(L, Q, G):
    T = L // G
    assert T * G == L

    def body(g_ref, init_ref, final_ref, out_ref, d_ref):
        r = lax.broadcasted_iota(jnp.int32, (G * Q, Q), 0)
        c = lax.broadcasted_iota(jnp.int32, (G * Q, Q), 1)
        d_ref[...] = jnp.where((r % Q) == c, 1.0, 0.0).astype(jnp.float32)

        def step(i, carry):
            for g in range(G):
                a = g_ref[pl.ds((g * T + i) * Q, Q), :]
                d = d_ref[g * Q:(g + 1) * Q, :]
                d_ref[g * Q:(g + 1) * Q, :] = jnp.dot(
                    d, a, preferred_element_type=jnp.float32)
            return carry

        lax.fori_loop(0, T, step, 0)

        x = init_ref[...]
        for g in range(G):
            x = jnp.dot(x, d_ref[g * Q:(g + 1) * Q, :],
                        preferred_element_type=jnp.float32)
        out_ref[...] = jnp.sum(x * final_ref[...], keepdims=True)

    return pl.pallas_call(
        body,
        out_shape=jax.ShapeDtypeStruct((1, 1), jnp.float32),
        scratch_shapes=[pltpu.VMEM((G * Q, Q), jnp.float32)],
    )


def kernel(A, input, init, final):
    Q, V, _ = A.shape
    L = input.shape[0]
    table = A.reshape(Q * V, Q)
    x = input.astype(jnp.int32)
    idx = (x[:, None] + (jnp.arange(Q, dtype=jnp.int32) * V)[None, :]).reshape(-1)
    gathered = _make_gather(Q * V, Q, L * Q)(table, idx)
    y = _make_chain(L, Q, 8)(gathered, init.reshape(1, Q), final.reshape(1, Q))
    return y.reshape(())


# speedup vs baseline: 22.2203x; 16.9754x over previous
import functools
import jax
import jax.numpy as jnp
from jax import lax
from jax.experimental import pallas as pl
from jax.experimental.pallas import tpu as pltpu
from jax.experimental.pallas import tpu_sc as plsc

NC, NS = 2, 16
NW = NC * NS


def _make_gather(Q, V, L):
    per_w = -(-L // NW)
    assert per_w <= 16
    H = Q // 2
    nfull = L // NW
    nextra = L - nfull * NW
    Lp = L + 32
    mesh = plsc.VectorSubcoreMesh(core_axis_name="c", subcore_axis_name="s")

    scratch = [
        pltpu.VMEM((Lp,), jnp.int32),
        pltpu.VMEM((Q, H, 128), jnp.float32),
        pltpu.VMEM((Q, Q), jnp.float32),
    ]

    @functools.partial(
        pl.kernel,
        out_type=jax.ShapeDtypeStruct((L * Q, Q), jnp.float32),
        mesh=mesh,
        scratch_types=scratch,
        compiler_params=pltpu.CompilerParams(needs_layout_passes=False),
    )
    def gather(ap_hbm, idx_hbm, out_hbm, idx_v, buf, out_v):
        wid = lax.axis_index("s") * NC + lax.axis_index("c")
        count = jnp.where(wid < nextra, nfull + 1, nfull)
        base = nfull * wid + jnp.minimum(wid, nextra)
        pltpu.sync_copy(idx_hbm, idx_v)
        chunk = idx_v[pl.ds(base, 16)]
        lanes = lax.iota(jnp.int32, 16)
        for k in range(per_w):
            @pl.when(k < count)
            def _():
                t = base + k
                s = chunk[k]
                u = pl.multiple_of((s // 128) * 128, 128)
                sloc = s - u
                for h in range(2):
                    pltpu.sync_copy(
                        ap_hbm.at[:, pl.ds(h * H, H), pl.ds(u, 128)], buf)
                    for q1 in range(Q):
                        vec = plsc.load_gather(
                            buf,
                            [jnp.full((16,), q1, jnp.int32),
                             lanes,
                             jnp.full((16,), sloc, jnp.int32)])
                        out_v[q1, pl.ds(h * H, H)] = vec
                pltpu.sync_copy(out_v, out_hbm.at[pl.ds(t * Q, Q), :])

    return gather, Lp


def _make_chain(L, Q, G):
    T = L // G
    assert T * G == L

    def body(g_ref, init_ref, final_ref, out_ref, d_ref):
        r = lax.broadcasted_iota(jnp.int32, (G * Q, Q), 0)
        c = lax.broadcasted_iota(jnp.int32, (G * Q, Q), 1)
        d_ref[...] = jnp.where((r % Q) == c, 1.0, 0.0).astype(jnp.float32)

        def step(i, carry):
            for g in range(G):
                a = g_ref[pl.ds((g * T + i) * Q, Q), :]
                d = d_ref[g * Q:(g + 1) * Q, :]
                d_ref[g * Q:(g + 1) * Q, :] = jnp.dot(
                    d, a, preferred_element_type=jnp.float32)
            return carry

        lax.fori_loop(0, T, step, 0)

        x = init_ref[...]
        for g in range(G):
            x = jnp.dot(x, d_ref[g * Q:(g + 1) * Q, :],
                        preferred_element_type=jnp.float32)
        out_ref[...] = jnp.sum(x * final_ref[...], keepdims=True)

    return pl.pallas_call(
        body,
        out_shape=jax.ShapeDtypeStruct((1, 1), jnp.float32),
        scratch_shapes=[pltpu.VMEM((G * Q, Q), jnp.float32)],
    )


def kernel(A, input, init, final):
    Q, V, _ = A.shape
    L = input.shape[0]
    ap = jnp.transpose(A, (0, 2, 1))
    x = input.astype(jnp.int32)
    gather, Lp = _make_gather(Q, V, L)
    xpad = jnp.zeros((Lp,), jnp.int32).at[:L].set(x)
    gathered = gather(ap, xpad)
    y = _make_chain(L, Q, 8)(gathered, init.reshape(1, Q), final.reshape(1, Q))
    return y.reshape(())
